# baseline (speedup 1.0000x reference)
import jax
import jax.numpy as jnp
from jax import lax
from jax.experimental import pallas as pl
from jax.experimental.pallas import tpu as pltpu
from jax.experimental.pallas import tpu_sc as plsc

T = 2048
H = 1024
HK = H // 2
DFF = 1408
E = 8
TOPK = 2
SDFF = 2 * DFF
NPAIR = T * TOPK
TILE = 256
NT_R = 23
MAXP = NT_R * TILE
NC = 2
NS = 16
NW = NC * NS
CH = 64


def _router_body(x_ref, gw_ref, d01_ref, w0_ref, w1_ref, te_ref):
    x = x_ref[...]
    gw = gw_ref[...]
    logits = lax.dot_general(x, gw, (((1,), (1,)), ((), ())),
                             preferred_element_type=jnp.float32)
    m = jnp.max(logits, axis=1, keepdims=True)
    ex = jnp.exp(logits - m)
    s = ex / jnp.sum(ex, axis=1, keepdims=True)
    iota8 = lax.broadcasted_iota(jnp.int32, (T, E), 1)
    m1 = jnp.max(s, axis=1, keepdims=True)
    i1 = jnp.min(jnp.where(s == m1, iota8, E), axis=1, keepdims=True)
    s2 = jnp.where(iota8 == i1, -1.0, s)
    m2 = jnp.max(s2, axis=1, keepdims=True)
    i2 = jnp.min(jnp.where(s2 == m2, iota8, E), axis=1, keepdims=True)
    tot = m1 + m2 + 1e-20
    w0_ref[...] = m1 / tot
    w1_ref[...] = m2 / tot
    oh0 = (iota8 == i1).astype(jnp.float32)
    oh1 = (iota8 == i2).astype(jnp.float32)
    rb = lax.broadcasted_iota(jnp.int32, (256, 256), 0)
    cb = lax.broadcasted_iota(jnp.int32, (256, 256), 1)
    ls = (rb > cb).astype(jnp.float32)
    carry = jnp.zeros((1, E), jnp.float32)
    rank = []
    for oh in (oh0, oh1):
        rk = []
        for b in range(T // 256):
            ohb = lax.slice(oh, (b * 256, 0), ((b + 1) * 256, E))
            wb = lax.dot_general(ls, ohb, (((1,), (0,)), ((), ())),
                                 preferred_element_type=jnp.float32)
            rk.append(jnp.sum((wb + carry) * ohb, axis=1, keepdims=True))
            carry = carry + jnp.sum(ohb, axis=0, keepdims=True)
        rank.append(jnp.concatenate(rk, axis=0))
    counts = carry
    pc = jnp.floor((counts + (TILE - 1.0)) / TILE) * TILE
    r8 = lax.broadcasted_iota(jnp.int32, (E, E), 0)
    c8 = lax.broadcasted_iota(jnp.int32, (E, E), 1)
    ul = (r8 <= c8).astype(jnp.float32)
    pe = lax.dot_general(pc, ul, (((1,), (0,)), ((), ())),
                         preferred_element_type=jnp.float32)
    po = pe - pc
    d0 = rank[0] + jnp.sum(oh0 * po, axis=1, keepdims=True)
    d1 = rank[1] + jnp.sum(oh1 * po, axis=1, keepdims=True)
    d01_ref[0:T, :] = d0.astype(jnp.int32)
    d01_ref[T:NPAIR, :] = d1.astype(jnp.int32)
    ti = lax.broadcasted_iota(jnp.int32, (1, 128), 1).astype(jnp.float32) * TILE
    acc = jnp.zeros((1, 128), jnp.float32)
    for e in range(E):
        acc = acc + (ti >= pe[0:1, e:e + 1]).astype(jnp.float32)
    iota_e = lax.broadcasted_iota(jnp.int32, (1, E), 1).astype(jnp.float32)
    emax = jnp.max(jnp.where(counts > 0.0, iota_e, 0.0), axis=1,
                   keepdims=True)
    te_ref[...] = jnp.minimum(acc, emax).astype(jnp.int32)


def _router(x, gate_w):
    return pl.pallas_call(
        _router_body,
        out_shape=(
            jax.ShapeDtypeStruct((NPAIR, 1), jnp.int32),
            jax.ShapeDtypeStruct((T, 1), jnp.float32),
            jax.ShapeDtypeStruct((T, 1), jnp.float32),
            jax.ShapeDtypeStruct((1, 128), jnp.int32),
        ),
    )(x, gate_w)


def _dispatch_body(x_hbm, d01_hbm, xs_hbm, idx_v, rows_v, sem):
    wid = lax.axis_index("s") * NC + lax.axis_index("c")
    per_w = NPAIR // NW
    for j in range(per_w // CH):
        base = wid * per_w + j * CH
        tbase = jnp.where(base >= T, base - T, base)
        pltpu.sync_copy(d01_hbm.at[pl.ds(base, CH)], idx_v)
        pltpu.sync_copy(x_hbm.at[pl.ds(tbase, CH)], rows_v)
        pltpu.async_copy(rows_v, xs_hbm.at[idx_v], sem).wait()


def _dispatch(x, d01):
    mesh = plsc.VectorSubcoreMesh(core_axis_name="c", subcore_axis_name="s")
    return pl.kernel(
        _dispatch_body,
        mesh=mesh,
        out_type=jax.ShapeDtypeStruct((MAXP, H), jnp.float32),
        scratch_types=[
            pltpu.VMEM((CH,), jnp.int32),
            pltpu.VMEM((CH, H), jnp.float32),
            pltpu.SemaphoreType.DMA,
        ],
    )(x, d01)


def _gmm_body(te_ref, xs_ref, up_ref, dn_ref, ys_ref, h_ref):
    k = pl.program_id(1)
    xk = xs_ref[...].astype(jnp.bfloat16)
    upk = up_ref[0].astype(jnp.bfloat16)
    ph = lax.dot_general(xk, upk, (((1,), (1,)), ((), ())),
                         preferred_element_type=jnp.float32)

    @pl.when(k == 0)
    def _():
        h_ref[...] = ph

    @pl.when(k == 1)
    def _():
        h = h_ref[...] + ph
        g = h[:, :DFF]
        u = h[:, DFF:]
        a = (g * lax.logistic(g) * u).astype(jnp.bfloat16)
        dn = dn_ref[0].astype(jnp.bfloat16)
        ys_ref[...] = lax.dot_general(a, dn, (((1,), (1,)), ((), ())),
                                      preferred_element_type=jnp.float32)


def _gmm(te, xs, up_w, down_w):
    return pl.pallas_call(
        _gmm_body,
        grid_spec=pltpu.PrefetchScalarGridSpec(
            num_scalar_prefetch=1,
            grid=(NT_R, 2),
            in_specs=[
                pl.BlockSpec((TILE, HK), lambda i, k, te: (i, k)),
                pl.BlockSpec((1, 2 * DFF, HK), lambda i, k, te: (te[i], 0, k)),
                pl.BlockSpec((1, H, DFF), lambda i, k, te: (te[i], 0, 0)),
            ],
            out_specs=pl.BlockSpec((TILE, H), lambda i, k, te: (i, 0)),
            scratch_shapes=[pltpu.VMEM((TILE, 2 * DFF), jnp.float32)],
        ),
        out_shape=jax.ShapeDtypeStruct((MAXP, H), jnp.float32),
    )(te, xs, up_w, down_w)


def _combine_body(ys_hbm, d01_hbm, yr_hbm, idx_v, rows_v, sem):
    wid = lax.axis_index("s") * NC + lax.axis_index("c")
    per_w = NPAIR // NW
    for j in range(per_w // CH):
        base = wid * per_w + j * CH
        pltpu.sync_copy(d01_hbm.at[pl.ds(base, CH)], idx_v)
        pltpu.async_copy(ys_hbm.at[idx_v], rows_v, sem).wait()
        pltpu.sync_copy(rows_v, yr_hbm.at[pl.ds(base, CH)])


def _combine(ys, d01):
    mesh = plsc.VectorSubcoreMesh(core_axis_name="c", subcore_axis_name="s")
    return pl.kernel(
        _combine_body,
        mesh=mesh,
        out_type=jax.ShapeDtypeStruct((NPAIR, H), jnp.float32),
        scratch_types=[
            pltpu.VMEM((CH,), jnp.int32),
            pltpu.VMEM((CH, H), jnp.float32),
            pltpu.SemaphoreType.DMA,
        ],
    )(ys, d01)


def _shared_body(x_ref, su_ref, sd_ref, y0_ref, y1_ref, w0_ref, w1_ref,
                 out_ref, h_ref):
    k = pl.program_id(1)
    xk = x_ref[...].astype(jnp.bfloat16)
    suk = su_ref[...].astype(jnp.bfloat16)
    ph = lax.dot_general(xk, suk, (((1,), (1,)), ((), ())),
                         preferred_element_type=jnp.float32)

    @pl.when(k == 0)
    def _():
        h_ref[...] = ph

    @pl.when(k == 1)
    def _():
        h = h_ref[...] + ph
        g = h[:, :SDFF]
        u = h[:, SDFF:]
        a = (g * lax.logistic(g) * u).astype(jnp.bfloat16)
        sd = sd_ref[...].astype(jnp.bfloat16)
        sh = lax.dot_general(a, sd, (((1,), (1,)), ((), ())),
                             preferred_element_type=jnp.float32)
        out_ref[...] = (sh + w0_ref[...] * y0_ref[...]
                        + w1_ref[...] * y1_ref[...])


def _shared_combine(x, shared_up_w, shared_down_w, yr, w0, w1):
    nt = T // TILE
    return pl.pallas_call(
        _shared_body,
        grid=(nt, 2),
        in_specs=[
            pl.BlockSpec((TILE, HK), lambda i, k: (i, k)),
            pl.BlockSpec((2 * SDFF, HK), lambda i, k: (0, k)),
            pl.BlockSpec((H, SDFF), lambda i, k: (0, 0),
                         pipeline_mode=pl.Buffered(buffer_count=1)),
            pl.BlockSpec((TILE, H), lambda i, k: (i, 0)),
            pl.BlockSpec((TILE, H), lambda i, k: (i + nt, 0)),
            pl.BlockSpec((TILE, 1), lambda i, k: (i, 0)),
            pl.BlockSpec((TILE, 1), lambda i, k: (i, 0)),
        ],
        out_specs=pl.BlockSpec((TILE, H), lambda i, k: (i, 0)),
        out_shape=jax.ShapeDtypeStruct((T, H), jnp.float32),
        scratch_shapes=[pltpu.VMEM((TILE, 2 * SDFF), jnp.float32)],
    )(x, shared_up_w, shared_down_w, yr, yr, w0, w1)


def kernel(x, gate_w, up_w, down_w, shared_up_w, shared_down_w):
    d01, w0, w1, te128 = _router(x, gate_w)
    d01f = d01.reshape(NPAIR)
    te = te128.reshape(128)[:NT_R]
    xs = _dispatch(x, d01f)
    ys = _gmm(te, xs, up_w, down_w)
    yr = _combine(ys, d01f)
    return _shared_combine(x, shared_up_w, shared_down_w, yr, w0, w1)


# speedup vs baseline: 1.2818x; 1.2818x over previous
import jax
import jax.numpy as jnp
from jax import lax
from jax.experimental import pallas as pl
from jax.experimental.pallas import tpu as pltpu
from jax.experimental.pallas import tpu_sc as plsc

T = 2048
H = 1024
DFF = 1408
E = 8
TOPK = 2
SDFF = 2 * DFF
NPAIR = T * TOPK
TILE = 256
NT_R = 23
MAXP = NT_R * TILE
NC = 2
NS = 16
NW = NC * NS
CH = 64


def _router_body(x_ref, gw_ref, d01_ref, w0_ref, w1_ref, te_ref):
    x = x_ref[...]
    gw = gw_ref[...]
    logits = lax.dot_general(x, gw, (((1,), (1,)), ((), ())),
                             preferred_element_type=jnp.float32)
    m = jnp.max(logits, axis=1, keepdims=True)
    ex = jnp.exp(logits - m)
    s = ex / jnp.sum(ex, axis=1, keepdims=True)
    iota8 = lax.broadcasted_iota(jnp.int32, (T, E), 1)
    m1 = jnp.max(s, axis=1, keepdims=True)
    i1 = jnp.min(jnp.where(s == m1, iota8, E), axis=1, keepdims=True)
    s2 = jnp.where(iota8 == i1, -1.0, s)
    m2 = jnp.max(s2, axis=1, keepdims=True)
    i2 = jnp.min(jnp.where(s2 == m2, iota8, E), axis=1, keepdims=True)
    tot = m1 + m2 + 1e-20
    w0_ref[...] = m1 / tot
    w1_ref[...] = m2 / tot
    oh0 = (iota8 == i1).astype(jnp.float32)
    oh1 = (iota8 == i2).astype(jnp.float32)
    rb = lax.broadcasted_iota(jnp.int32, (256, 256), 0)
    cb = lax.broadcasted_iota(jnp.int32, (256, 256), 1)
    ls = (rb > cb).astype(jnp.float32)
    carry = jnp.zeros((1, E), jnp.float32)
    rank = []
    for oh in (oh0, oh1):
        rk = []
        for b in range(T // 256):
            ohb = lax.slice(oh, (b * 256, 0), ((b + 1) * 256, E))
            wb = lax.dot_general(ls, ohb, (((1,), (0,)), ((), ())),
                                 preferred_element_type=jnp.float32)
            rk.append(jnp.sum((wb + carry) * ohb, axis=1, keepdims=True))
            carry = carry + jnp.sum(ohb, axis=0, keepdims=True)
        rank.append(jnp.concatenate(rk, axis=0))
    counts = carry
    pc = jnp.floor((counts + (TILE - 1.0)) / TILE) * TILE
    r8 = lax.broadcasted_iota(jnp.int32, (E, E), 0)
    c8 = lax.broadcasted_iota(jnp.int32, (E, E), 1)
    ul = (r8 <= c8).astype(jnp.float32)
    pe = lax.dot_general(pc, ul, (((1,), (0,)), ((), ())),
                         preferred_element_type=jnp.float32)
    po = pe - pc
    d0 = rank[0] + jnp.sum(oh0 * po, axis=1, keepdims=True)
    d1 = rank[1] + jnp.sum(oh1 * po, axis=1, keepdims=True)
    d01_ref[0:T, :] = d0.astype(jnp.int32)
    d01_ref[T:NPAIR, :] = d1.astype(jnp.int32)
    ti = lax.broadcasted_iota(jnp.int32, (1, 128), 1).astype(jnp.float32) * TILE
    acc = jnp.zeros((1, 128), jnp.float32)
    for e in range(E):
        acc = acc + (ti >= pe[0:1, e:e + 1]).astype(jnp.float32)
    iota_e = lax.broadcasted_iota(jnp.int32, (1, E), 1).astype(jnp.float32)
    emax = jnp.max(jnp.where(counts > 0.0, iota_e, 0.0), axis=1,
                   keepdims=True)
    te_ref[...] = jnp.minimum(acc, emax).astype(jnp.int32)


def _router(x, gate_w):
    return pl.pallas_call(
        _router_body,
        out_shape=(
            jax.ShapeDtypeStruct((NPAIR, 1), jnp.int32),
            jax.ShapeDtypeStruct((T, 1), jnp.float32),
            jax.ShapeDtypeStruct((T, 1), jnp.float32),
            jax.ShapeDtypeStruct((1, 128), jnp.int32),
        ),
    )(x, gate_w)


def _dispatch_body(x_hbm, d01_hbm, xs_hbm, idx_v, rows_v, sem):
    wid = lax.axis_index("s") * NC + lax.axis_index("c")
    per_w = NPAIR // NW
    for j in range(per_w // CH):
        base = wid * per_w + j * CH
        tbase = jnp.where(base >= T, base - T, base)
        pltpu.sync_copy(d01_hbm.at[pl.ds(base, CH)], idx_v)
        pltpu.sync_copy(x_hbm.at[pl.ds(tbase, CH)], rows_v)
        pltpu.async_copy(rows_v, xs_hbm.at[idx_v], sem).wait()


def _dispatch(x, d01):
    mesh = plsc.VectorSubcoreMesh(core_axis_name="c", subcore_axis_name="s")
    return pl.kernel(
        _dispatch_body,
        mesh=mesh,
        out_type=jax.ShapeDtypeStruct((MAXP, H), jnp.float32),
        scratch_types=[
            pltpu.VMEM((CH,), jnp.int32),
            pltpu.VMEM((CH, H), jnp.float32),
            pltpu.SemaphoreType.DMA,
        ],
    )(x, d01)


def _gmm_body(te_ref, xs_ref, up_ref, dn_ref, ys_ref):
    xt = xs_ref[...].astype(jnp.bfloat16)
    up = up_ref[0].astype(jnp.bfloat16)
    h = lax.dot_general(xt, up, (((1,), (1,)), ((), ())),
                        preferred_element_type=jnp.float32)
    g = h[:, :DFF]
    u = h[:, DFF:]
    a = (g * lax.logistic(g) * u).astype(jnp.bfloat16)
    dn = dn_ref[0].astype(jnp.bfloat16)
    ys_ref[...] = lax.dot_general(a, dn, (((1,), (1,)), ((), ())),
                                  preferred_element_type=jnp.float32)


def _gmm(te, xs, up_w, down_w):
    return pl.pallas_call(
        _gmm_body,
        grid_spec=pltpu.PrefetchScalarGridSpec(
            num_scalar_prefetch=1,
            grid=(NT_R,),
            in_specs=[
                pl.BlockSpec((TILE, H), lambda i, te: (i, 0)),
                pl.BlockSpec((1, 2 * DFF, H), lambda i, te: (te[i], 0, 0)),
                pl.BlockSpec((1, H, DFF), lambda i, te: (te[i], 0, 0)),
            ],
            out_specs=pl.BlockSpec((TILE, H), lambda i, te: (i, 0)),
        ),
        out_shape=jax.ShapeDtypeStruct((MAXP, H), jnp.float32),
    )(te, xs, up_w, down_w)


def _combine_body(ys_hbm, d01_hbm, yr_hbm, idx_v, rows_v, sem):
    wid = lax.axis_index("s") * NC + lax.axis_index("c")
    per_w = NPAIR // NW
    for j in range(per_w // CH):
        base = wid * per_w + j * CH
        pltpu.sync_copy(d01_hbm.at[pl.ds(base, CH)], idx_v)
        pltpu.async_copy(ys_hbm.at[idx_v], rows_v, sem).wait()
        pltpu.sync_copy(rows_v, yr_hbm.at[pl.ds(base, CH)])


def _combine(ys, d01):
    mesh = plsc.VectorSubcoreMesh(core_axis_name="c", subcore_axis_name="s")
    return pl.kernel(
        _combine_body,
        mesh=mesh,
        out_type=jax.ShapeDtypeStruct((NPAIR, H), jnp.float32),
        scratch_types=[
            pltpu.VMEM((CH,), jnp.int32),
            pltpu.VMEM((CH, H), jnp.float32),
            pltpu.SemaphoreType.DMA,
        ],
    )(ys, d01)


def _shared_body(x_ref, su_ref, sd_ref, y0_ref, y1_ref, w0_ref, w1_ref,
                 out_ref):
    xt = x_ref[...].astype(jnp.bfloat16)
    su = su_ref[...].astype(jnp.bfloat16)
    h = lax.dot_general(xt, su, (((1,), (1,)), ((), ())),
                        preferred_element_type=jnp.float32)
    g = h[:, :SDFF]
    u = h[:, SDFF:]
    a = (g * lax.logistic(g) * u).astype(jnp.bfloat16)
    sd = sd_ref[...].astype(jnp.bfloat16)
    sh = lax.dot_general(a, sd, (((1,), (1,)), ((), ())),
                         preferred_element_type=jnp.float32)
    out_ref[...] = sh + w0_ref[...] * y0_ref[...] + w1_ref[...] * y1_ref[...]


def _shared_combine(x, shared_up_w, shared_down_w, yr, w0, w1):
    nt = T // TILE
    return pl.pallas_call(
        _shared_body,
        grid=(nt,),
        in_specs=[
            pl.BlockSpec((TILE, H), lambda i: (i, 0)),
            pl.BlockSpec((2 * SDFF, H), lambda i: (0, 0)),
            pl.BlockSpec((H, SDFF), lambda i: (0, 0)),
            pl.BlockSpec((TILE, H), lambda i: (i, 0)),
            pl.BlockSpec((TILE, H), lambda i: (i + nt, 0)),
            pl.BlockSpec((TILE, 1), lambda i: (i, 0)),
            pl.BlockSpec((TILE, 1), lambda i: (i, 0)),
        ],
        out_specs=pl.BlockSpec((TILE, H), lambda i: (i, 0)),
        out_shape=jax.ShapeDtypeStruct((T, H), jnp.float32),
    )(x, shared_up_w, shared_down_w, yr, yr, w0, w1)


def kernel(x, gate_w, up_w, down_w, shared_up_w, shared_down_w):
    d01, w0, w1, te128 = _router(x, gate_w)
    d01f = d01.reshape(NPAIR)
    te = te128.reshape(128)[:NT_R]
    xs = _dispatch(x, d01f)
    ys = _gmm(te, xs, up_w, down_w)
    yr = _combine(ys, d01f)
    return _shared_combine(x, shared_up_w, shared_down_w, yr, w0, w1)


# speedup vs baseline: 1.3527x; 1.0553x over previous
import jax
import jax.numpy as jnp
from jax import lax
from jax.experimental import pallas as pl
from jax.experimental.pallas import tpu as pltpu
from jax.experimental.pallas import tpu_sc as plsc

T = 2048
H = 1024
DFF = 1408
E = 8
TOPK = 2
SDFF = 2 * DFF
NPAIR = T * TOPK
TILE = 256
NT_R = 23
MAXP = NT_R * TILE
NC = 2
NS = 16
NW = NC * NS
CH = 64


def _router_body(x_ref, gw_ref, d01_ref, w0_ref, w1_ref, te_ref):
    x = x_ref[...]
    gw = gw_ref[...]
    logits = lax.dot_general(x, gw, (((1,), (1,)), ((), ())),
                             preferred_element_type=jnp.float32)
    m = jnp.max(logits, axis=1, keepdims=True)
    ex = jnp.exp(logits - m)
    s = ex / jnp.sum(ex, axis=1, keepdims=True)
    iota8 = lax.broadcasted_iota(jnp.int32, (T, E), 1)
    m1 = jnp.max(s, axis=1, keepdims=True)
    i1 = jnp.min(jnp.where(s == m1, iota8, E), axis=1, keepdims=True)
    s2 = jnp.where(iota8 == i1, -1.0, s)
    m2 = jnp.max(s2, axis=1, keepdims=True)
    i2 = jnp.min(jnp.where(s2 == m2, iota8, E), axis=1, keepdims=True)
    tot = m1 + m2 + 1e-20
    w0_ref[...] = m1 / tot
    w1_ref[...] = m2 / tot
    oh0 = (iota8 == i1).astype(jnp.float32)
    oh1 = (iota8 == i2).astype(jnp.float32)
    rb = lax.broadcasted_iota(jnp.int32, (256, 256), 0)
    cb = lax.broadcasted_iota(jnp.int32, (256, 256), 1)
    ls = (rb > cb).astype(jnp.float32)
    carry = jnp.zeros((1, E), jnp.float32)
    rank = []
    for oh in (oh0, oh1):
        rk = []
        for b in range(T // 256):
            ohb = lax.slice(oh, (b * 256, 0), ((b + 1) * 256, E))
            wb = lax.dot_general(ls, ohb, (((1,), (0,)), ((), ())),
                                 preferred_element_type=jnp.float32)
            rk.append(jnp.sum((wb + carry) * ohb, axis=1, keepdims=True))
            carry = carry + jnp.sum(ohb, axis=0, keepdims=True)
        rank.append(jnp.concatenate(rk, axis=0))
    counts = carry
    pc = jnp.floor((counts + (TILE - 1.0)) / TILE) * TILE
    r8 = lax.broadcasted_iota(jnp.int32, (E, E), 0)
    c8 = lax.broadcasted_iota(jnp.int32, (E, E), 1)
    ul = (r8 <= c8).astype(jnp.float32)
    pe = lax.dot_general(pc, ul, (((1,), (0,)), ((), ())),
                         preferred_element_type=jnp.float32)
    po = pe - pc
    d0 = rank[0] + jnp.sum(oh0 * po, axis=1, keepdims=True)
    d1 = rank[1] + jnp.sum(oh1 * po, axis=1, keepdims=True)
    d01_ref[0:T, :] = d0.astype(jnp.int32)
    d01_ref[T:NPAIR, :] = d1.astype(jnp.int32)
    ti = lax.broadcasted_iota(jnp.int32, (1, 128), 1).astype(jnp.float32) * TILE
    acc = jnp.zeros((1, 128), jnp.float32)
    for e in range(E):
        acc = acc + (ti >= pe[0:1, e:e + 1]).astype(jnp.float32)
    iota_e = lax.broadcasted_iota(jnp.int32, (1, E), 1).astype(jnp.float32)
    emax = jnp.max(jnp.where(counts > 0.0, iota_e, 0.0), axis=1,
                   keepdims=True)
    te = jnp.minimum(acc, emax)
    nt_used = pe[0:1, E - 1:E] / TILE
    i128 = lax.broadcasted_iota(jnp.int32, (1, 128), 1)
    te_ref[...] = jnp.where(i128 == NT_R, nt_used, te).astype(jnp.int32)


def _router(x, gate_w):
    return pl.pallas_call(
        _router_body,
        out_shape=(
            jax.ShapeDtypeStruct((NPAIR, 1), jnp.int32),
            jax.ShapeDtypeStruct((T, 1), jnp.float32),
            jax.ShapeDtypeStruct((T, 1), jnp.float32),
            jax.ShapeDtypeStruct((1, 128), jnp.int32),
        ),
    )(x, gate_w)


def _dispatch_body(x_hbm, d01_hbm, xs_hbm, idx0, idx1, rows_v, sem):
    wid = lax.axis_index("s") * NC + lax.axis_index("c")
    tok_w = T // NW
    base = wid * tok_w
    pltpu.sync_copy(x_hbm.at[pl.ds(base, tok_w)], rows_v)
    pltpu.sync_copy(d01_hbm.at[pl.ds(base, tok_w)], idx0)
    c0 = pltpu.async_copy(rows_v, xs_hbm.at[idx0], sem)
    pltpu.sync_copy(d01_hbm.at[pl.ds(T + base, tok_w)], idx1)
    c1 = pltpu.async_copy(rows_v, xs_hbm.at[idx1], sem)
    c0.wait()
    c1.wait()


def _dispatch(x, d01):
    mesh = plsc.VectorSubcoreMesh(core_axis_name="c", subcore_axis_name="s")
    tok_w = T // NW
    return pl.kernel(
        _dispatch_body,
        mesh=mesh,
        out_type=jax.ShapeDtypeStruct((MAXP, H), jnp.float32),
        scratch_types=[
            pltpu.VMEM((tok_w,), jnp.int32),
            pltpu.VMEM((tok_w,), jnp.int32),
            pltpu.VMEM((tok_w, H), jnp.float32),
            pltpu.SemaphoreType.DMA,
        ],
    )(x, d01)


def _gmm_body(te_ref, xs_ref, up_ref, dn_ref, ys_ref):
    @pl.when(pl.program_id(0) < te_ref[NT_R])
    def _():
        xt = xs_ref[...].astype(jnp.bfloat16)
        up = up_ref[0].astype(jnp.bfloat16)
        h = lax.dot_general(xt, up, (((1,), (1,)), ((), ())),
                            preferred_element_type=jnp.float32)
        g = h[:, :DFF]
        u = h[:, DFF:]
        a = (g * lax.logistic(g) * u).astype(jnp.bfloat16)
        dn = dn_ref[0].astype(jnp.bfloat16)
        ys_ref[...] = lax.dot_general(a, dn, (((1,), (1,)), ((), ())),
                                      preferred_element_type=jnp.float32)


def _gmm(te, xs, up_w, down_w):
    return pl.pallas_call(
        _gmm_body,
        grid_spec=pltpu.PrefetchScalarGridSpec(
            num_scalar_prefetch=1,
            grid=(NT_R,),
            in_specs=[
                pl.BlockSpec((TILE, H), lambda i, te: (i, 0)),
                pl.BlockSpec((1, 2 * DFF, H), lambda i, te: (te[i], 0, 0)),
                pl.BlockSpec((1, H, DFF), lambda i, te: (te[i], 0, 0)),
            ],
            out_specs=pl.BlockSpec((TILE, H), lambda i, te: (i, 0)),
        ),
        out_shape=jax.ShapeDtypeStruct((MAXP, H), jnp.float32),
    )(te, xs, up_w, down_w)


def _combine_body(ys_hbm, d01_hbm, yr_hbm, idx_v, rows_v, sem):
    wid = lax.axis_index("s") * NC + lax.axis_index("c")
    per_w = NPAIR // NW
    for j in range(per_w // CH):
        base = wid * per_w + j * CH
        pltpu.sync_copy(d01_hbm.at[pl.ds(base, CH)], idx_v)
        pltpu.async_copy(ys_hbm.at[idx_v], rows_v, sem).wait()
        pltpu.sync_copy(rows_v, yr_hbm.at[pl.ds(base, CH)])


def _combine(ys, d01):
    mesh = plsc.VectorSubcoreMesh(core_axis_name="c", subcore_axis_name="s")
    return pl.kernel(
        _combine_body,
        mesh=mesh,
        out_type=jax.ShapeDtypeStruct((NPAIR, H), jnp.float32),
        scratch_types=[
            pltpu.VMEM((CH,), jnp.int32),
            pltpu.VMEM((CH, H), jnp.float32),
            pltpu.SemaphoreType.DMA,
        ],
    )(ys, d01)


def _shared_body(x_ref, su_ref, sd_ref, y0_ref, y1_ref, w0_ref, w1_ref,
                 out_ref):
    xt = x_ref[...].astype(jnp.bfloat16)
    su = su_ref[...].astype(jnp.bfloat16)
    h = lax.dot_general(xt, su, (((1,), (1,)), ((), ())),
                        preferred_element_type=jnp.float32)
    g = h[:, :SDFF]
    u = h[:, SDFF:]
    a = (g * lax.logistic(g) * u).astype(jnp.bfloat16)
    sd = sd_ref[...].astype(jnp.bfloat16)
    sh = lax.dot_general(a, sd, (((1,), (1,)), ((), ())),
                         preferred_element_type=jnp.float32)
    out_ref[...] = sh + w0_ref[...] * y0_ref[...] + w1_ref[...] * y1_ref[...]


def _shared_combine(x, shared_up_w, shared_down_w, yr, w0, w1):
    nt = T // TILE
    return pl.pallas_call(
        _shared_body,
        grid=(nt,),
        in_specs=[
            pl.BlockSpec((TILE, H), lambda i: (i, 0)),
            pl.BlockSpec((2 * SDFF, H), lambda i: (0, 0)),
            pl.BlockSpec((H, SDFF), lambda i: (0, 0)),
            pl.BlockSpec((TILE, H), lambda i: (i, 0)),
            pl.BlockSpec((TILE, H), lambda i: (i + nt, 0)),
            pl.BlockSpec((TILE, 1), lambda i: (i, 0)),
            pl.BlockSpec((TILE, 1), lambda i: (i, 0)),
        ],
        out_specs=pl.BlockSpec((TILE, H), lambda i: (i, 0)),
        out_shape=jax.ShapeDtypeStruct((T, H), jnp.float32),
    )(x, shared_up_w, shared_down_w, yr, yr, w0, w1)


def kernel(x, gate_w, up_w, down_w, shared_up_w, shared_down_w):
    d01, w0, w1, te128 = _router(x, gate_w)
    d01f = d01.reshape(NPAIR)
    te = te128.reshape(128)[:NT_R + 1]
    xs = _dispatch(x, d01f)
    ys = _gmm(te, xs, up_w, down_w)
    yr = _combine(ys, d01f)
    return _shared_combine(x, shared_up_w, shared_down_w, yr, w0, w1)
